# baseline (speedup 1.0000x reference)
import jax
import jax.numpy as jnp
from jax import lax
from jax.experimental import pallas as pl
from jax.experimental.pallas import tpu as pltpu
from jax.experimental.pallas import tpu_sc as plsc

NC = 2
NS = 16
NW = NC * NS

BATCH = 4096
SEQ = 200
D_MODEL = 64
NPAIR = 500000
BBLK = BATCH // NW
NBUF = 4
N_GROUPS = SEQ // NBUF


def _gather_body(idxT_hbm, table2_hbm, outT_hbm, idx_v, pidx_v, off_v,
                 rows_v, trans_v, gsems, ssems):
    t = lax.axis_index("s") * NC + lax.axis_index("c")
    b0 = t * BBLK

    pltpu.sync_copy(idxT_hbm.at[:, pl.ds(b0, BBLK)], idx_v)

    iota = lax.iota(jnp.int32, 16)

    def compute_pidx(s, b):
        for k in range(BBLK // 16):
            v = idx_v[s, pl.ds(16 * k, 16)]
            pidx_v[b, pl.ds(16 * k, 16)] = lax.shift_right_logical(v, 1)

    def gather_copy(b):
        return pltpu.make_async_copy(
            table2_hbm.at[pidx_v.at[b]], rows_v.at[b], gsems.at[b])

    def store_copy(s, b):
        dst = outT_hbm.at[s].at[:, pl.ds(b0, BBLK)]
        return pltpu.make_async_copy(trans_v.at[b], dst, ssems.at[b])

    def transpose(s, b):
        for rb in range(BBLK // 16):
            off_v[b, pl.ds(16 * rb, 16)] = lax.shift_left(
                lax.bitwise_and(idx_v[s, pl.ds(16 * rb, 16)], 1), 6)

        @plsc.parallel_loop(0, D_MODEL, 1, unroll=8)
        def frow(f):
            for rb in range(BBLK // 16):
                offs = off_v[b, pl.ds(16 * rb, 16)]
                vals = plsc.load_gather(
                    rows_v.at[b], [iota + 16 * rb, offs + f])
                trans_v[b, f, pl.ds(16 * rb, 16)] = vals

    for b in range(NBUF):
        compute_pidx(b, b)
        gather_copy(b).start()

    for b in range(NBUF):
        s = b
        gather_copy(b).wait()
        transpose(s, b)
        store_copy(s, b).start()
        compute_pidx(s + NBUF, b)
        gather_copy(b).start()

    def group(g, _):
        for b in range(NBUF):
            s = g * NBUF + b
            gather_copy(b).wait()
            store_copy(s - NBUF, b).wait()
            transpose(s, b)
            store_copy(s, b).start()
            compute_pidx(s + NBUF, b)
            gather_copy(b).start()
        return ()

    lax.fori_loop(1, N_GROUPS - 1, group, ())

    for b in range(NBUF):
        s = (N_GROUPS - 1) * NBUF + b
        gather_copy(b).wait()
        store_copy(s - NBUF, b).wait()
        transpose(s, b)
        store_copy(s, b).start()
    for b in range(NBUF):
        s = (N_GROUPS - 1) * NBUF + b
        store_copy(s, b).wait()


@jax.jit
def _embed(indices_t, table2):
    mesh = plsc.VectorSubcoreMesh(core_axis_name="c", subcore_axis_name="s")
    f = pl.kernel(
        _gather_body,
        out_type=jax.ShapeDtypeStruct((SEQ, D_MODEL, BATCH), jnp.float32),
        mesh=mesh,
        scratch_types=[
            pltpu.VMEM((SEQ, BBLK), jnp.int32),
            pltpu.VMEM((NBUF, BBLK), jnp.int32),
            pltpu.VMEM((NBUF, BBLK), jnp.int32),
            pltpu.VMEM((NBUF, BBLK, 128), jnp.float32),
            pltpu.VMEM((NBUF, D_MODEL, BBLK), jnp.float32),
            pltpu.SemaphoreType.DMA((NBUF,)),
            pltpu.SemaphoreType.DMA((NBUF,)),
        ],
        compiler_params=pltpu.CompilerParams(needs_layout_passes=False),
    )
    return f(indices_t, table2)


def kernel(indices, table):
    idx_t = indices.T.astype(jnp.int32)
    table2 = table.reshape(NPAIR, 128)
    out_t = _embed(idx_t, table2)
    return jnp.transpose(out_t, (2, 0, 1))


# speedup vs baseline: 1.5646x; 1.5646x over previous
import jax
import jax.numpy as jnp
from jax import lax
from jax.experimental import pallas as pl
from jax.experimental.pallas import tpu as pltpu
from jax.experimental.pallas import tpu_sc as plsc

NC = 2
NS = 16
NW = NC * NS

BATCH = 4096
SEQ = 200
D_MODEL = 64
NPAIR = 500000
BBLK = BATCH // NW
NBUF = 4
N_GROUPS = SEQ // NBUF


def _gather_body(idxT_hbm, table2_hbm, outT_hbm, idx_v, pidx_v, off_v,
                 rows_v, trans_v, gsems, ssems):
    t = lax.axis_index("s") * NC + lax.axis_index("c")
    b0 = t * BBLK

    pltpu.sync_copy(idxT_hbm.at[:, pl.ds(b0, BBLK)], idx_v)

    iota = lax.iota(jnp.int32, 16)

    def compute_pidx(s, b):
        for k in range(BBLK // 16):
            v = idx_v[s, pl.ds(16 * k, 16)]
            pidx_v[b, pl.ds(16 * k, 16)] = lax.shift_right_logical(v, 1)

    def gather_copy(b):
        return pltpu.make_async_copy(
            table2_hbm.at[pidx_v.at[b]], rows_v.at[b], gsems.at[b])

    def store_copy(s, b):
        dst = outT_hbm.at[s].at[:, pl.ds(b0, BBLK)]
        return pltpu.make_async_copy(trans_v.at[b], dst, ssems.at[b])

    def transpose(s, b):
        for rb in range(BBLK // 16):
            off_v[b, pl.ds(16 * rb, 16)] = lax.shift_left(
                lax.bitwise_and(idx_v[s, pl.ds(16 * rb, 16)], 1), 6)

        @plsc.parallel_loop(0, D_MODEL, 1, unroll=8)
        def frow(f):
            feat = lax.bitwise_and(f + iota, D_MODEL - 1)
            for rb in range(BBLK // 16):
                rows = iota + 16 * rb
                offs = off_v[b, pl.ds(16 * rb, 16)]
                vals = plsc.load_gather(rows_v.at[b], [rows, offs + feat])
                plsc.store_scatter(trans_v.at[b], [feat, rows], vals)

    for b in range(NBUF):
        compute_pidx(b, b)
        gather_copy(b).start()

    for b in range(NBUF):
        s = b
        gather_copy(b).wait()
        transpose(s, b)
        store_copy(s, b).start()
        compute_pidx(s + NBUF, b)
        gather_copy(b).start()

    def group(g, _):
        for b in range(NBUF):
            s = g * NBUF + b
            gather_copy(b).wait()
            store_copy(s - NBUF, b).wait()
            transpose(s, b)
            store_copy(s, b).start()
            compute_pidx(s + NBUF, b)
            gather_copy(b).start()
        return ()

    lax.fori_loop(1, N_GROUPS - 1, group, ())

    for b in range(NBUF):
        s = (N_GROUPS - 1) * NBUF + b
        gather_copy(b).wait()
        store_copy(s - NBUF, b).wait()
        transpose(s, b)
        store_copy(s, b).start()
    for b in range(NBUF):
        s = (N_GROUPS - 1) * NBUF + b
        store_copy(s, b).wait()


@jax.jit
def _embed(indices_t, table2):
    mesh = plsc.VectorSubcoreMesh(core_axis_name="c", subcore_axis_name="s")
    f = pl.kernel(
        _gather_body,
        out_type=jax.ShapeDtypeStruct((SEQ, D_MODEL, BATCH), jnp.float32),
        mesh=mesh,
        scratch_types=[
            pltpu.VMEM((SEQ, BBLK), jnp.int32),
            pltpu.VMEM((NBUF, BBLK), jnp.int32),
            pltpu.VMEM((NBUF, BBLK), jnp.int32),
            pltpu.VMEM((NBUF, BBLK, 128), jnp.float32),
            pltpu.VMEM((NBUF, D_MODEL, BBLK), jnp.float32),
            pltpu.SemaphoreType.DMA((NBUF,)),
            pltpu.SemaphoreType.DMA((NBUF,)),
        ],
        compiler_params=pltpu.CompilerParams(needs_layout_passes=False),
    )
    return f(indices_t, table2)


def kernel(indices, table):
    idx_t = indices.T.astype(jnp.int32)
    table2 = table.reshape(NPAIR, 128)
    out_t = _embed(idx_t, table2)
    return jnp.transpose(out_t, (2, 0, 1))


# speedup vs baseline: 2.0092x; 1.2842x over previous
import jax
import jax.numpy as jnp
from jax import lax
from jax.experimental import pallas as pl
from jax.experimental.pallas import tpu as pltpu
from jax.experimental.pallas import tpu_sc as plsc

NC = 2
NS = 16
NW = NC * NS

BATCH = 4096
SEQ = 200
D_MODEL = 64
NPAIR = 500000
BBLK = BATCH // NW
NBUF = 4
N_GROUPS = SEQ // NBUF


NFULL = 1000000 // 128
NBLK_MAIN = NFULL // NW
NTAIL = NFULL - NBLK_MAIN * NW
TNBUF = 3


def _transpose_body(tt_hbm, tail_hbm, t2_hbm, slab_v, pair_v, gsems, ssems):
    t = lax.axis_index("s") * NC + lax.axis_index("c")
    iota = lax.iota(jnp.int32, 16)
    two_q = tuple(2 * (16 * qb + iota) for qb in range(4))
    qvs = tuple(16 * qb + iota for qb in range(4))

    def load_copy(ig, b):
        src = tt_hbm.at[:, pl.ds(ig * 128, 128)]
        return pltpu.make_async_copy(src, slab_v.at[b], gsems.at[b])

    def store_copy(ig, b):
        dst = t2_hbm.at[pl.ds(ig * 64, 64)]
        return pltpu.make_async_copy(pair_v.at[b], dst, ssems.at[b])

    def transpose(b, nqb):
        @plsc.parallel_loop(0, 128, 1, unroll=4)
        def fcol(c2):
            c2v = lax.bitwise_and(c2 + iota, 127)
            fv = lax.bitwise_and(c2v, 63)
            hv = lax.shift_right_logical(c2v, 6)
            for qb in range(nqb):
                vals = plsc.load_gather(slab_v.at[b], [fv, two_q[qb] + hv])
                plsc.store_scatter(pair_v.at[b], [qvs[qb], c2v], vals)

    def my_ig(n):
        return n * NW + t

    for b in range(TNBUF):
        load_copy(my_ig(b), b).start()

    for b in range(TNBUF):
        n = b
        load_copy(my_ig(n), b).wait()
        transpose(b, 4)
        store_copy(my_ig(n), b).start()
        load_copy(my_ig(n + TNBUF), b).start()

    def loop(n, _):
        for b in range(TNBUF):
            i = n * TNBUF + b
            load_copy(my_ig(i), b).wait()
            store_copy(my_ig(i - TNBUF), b).wait()
            transpose(b, 4)
            store_copy(my_ig(i), b).start()
            load_copy(my_ig(i + TNBUF), b).start()
        return ()

    lax.fori_loop(1, NBLK_MAIN // TNBUF - 1, loop, ())

    for b in range(TNBUF):
        i = (NBLK_MAIN // TNBUF - 1) * TNBUF + b
        load_copy(my_ig(i), b).wait()
        store_copy(my_ig(i - TNBUF), b).wait()
        transpose(b, 4)
        store_copy(my_ig(i), b).start()

    rem = NBLK_MAIN - (NBLK_MAIN // TNBUF) * TNBUF
    for b in range(rem):
        i = (NBLK_MAIN // TNBUF) * TNBUF + b
        load_copy(my_ig(i), b).start()
        load_copy(my_ig(i), b).wait()
        store_copy(my_ig(i - TNBUF), b).wait()
        transpose(b, 4)
        store_copy(my_ig(i), b).start()
    for b in range(TNBUF):
        i = NBLK_MAIN - TNBUF + b
        store_copy(my_ig(i), b).wait()

    @pl.when(t < NTAIL)
    def _():
        ig = NBLK_MAIN * NW + t
        load_copy(ig, 0).start()
        load_copy(ig, 0).wait()
        transpose(0, 4)
        store_copy(ig, 0).start()
        store_copy(ig, 0).wait()

    @pl.when(t == NTAIL)
    def _():
        pltpu.sync_copy(tail_hbm, slab_v.at[0])

        @plsc.parallel_loop(0, 128, 1, unroll=4)
        def fcol(c2):
            c2v = lax.bitwise_and(c2 + iota, 127)
            fv = lax.bitwise_and(c2v, 63)
            hv = lax.shift_right_logical(c2v, 6)
            for qb in range(2):
                vals = plsc.load_gather(
                    slab_v.at[0], [two_q[qb] + hv, fv])
                plsc.store_scatter(pair_v.at[0], [qvs[qb], c2v], vals)

        pltpu.sync_copy(pair_v.at[0].at[pl.ds(0, 32)],
                        t2_hbm.at[pl.ds(NFULL * 64, 32)])


def _gather_body(idxT_hbm, table2_hbm, outT_hbm, idx_v, pidx_v, off_v,
                 rows_v, trans_v, gsems, ssems):
    t = lax.axis_index("s") * NC + lax.axis_index("c")
    b0 = t * BBLK

    pltpu.sync_copy(idxT_hbm.at[:, pl.ds(b0, BBLK)], idx_v)

    iota = lax.iota(jnp.int32, 16)

    def compute_pidx(s, b):
        for k in range(BBLK // 16):
            v = idx_v[s, pl.ds(16 * k, 16)]
            pidx_v[b, pl.ds(16 * k, 16)] = lax.shift_right_logical(v, 1)

    def gather_copy(b):
        return pltpu.make_async_copy(
            table2_hbm.at[pidx_v.at[b]], rows_v.at[b], gsems.at[b])

    def store_copy(s, b):
        dst = outT_hbm.at[s].at[:, pl.ds(b0, BBLK)]
        return pltpu.make_async_copy(trans_v.at[b], dst, ssems.at[b])

    def transpose(s, b):
        for rb in range(BBLK // 16):
            off_v[b, pl.ds(16 * rb, 16)] = lax.shift_left(
                lax.bitwise_and(idx_v[s, pl.ds(16 * rb, 16)], 1), 6)

        @plsc.parallel_loop(0, D_MODEL, 1, unroll=8)
        def frow(f):
            feat = lax.bitwise_and(f + iota, D_MODEL - 1)
            for rb in range(BBLK // 16):
                rows = iota + 16 * rb
                offs = off_v[b, pl.ds(16 * rb, 16)]
                vals = plsc.load_gather(rows_v.at[b], [rows, offs + feat])
                plsc.store_scatter(trans_v.at[b], [feat, rows], vals)

    for b in range(NBUF):
        compute_pidx(b, b)
        gather_copy(b).start()

    for b in range(NBUF):
        s = b
        gather_copy(b).wait()
        transpose(s, b)
        store_copy(s, b).start()
        compute_pidx(s + NBUF, b)
        gather_copy(b).start()

    def group(g, _):
        for b in range(NBUF):
            s = g * NBUF + b
            gather_copy(b).wait()
            store_copy(s - NBUF, b).wait()
            transpose(s, b)
            store_copy(s, b).start()
            compute_pidx(s + NBUF, b)
            gather_copy(b).start()
        return ()

    lax.fori_loop(1, N_GROUPS - 1, group, ())

    for b in range(NBUF):
        s = (N_GROUPS - 1) * NBUF + b
        gather_copy(b).wait()
        store_copy(s - NBUF, b).wait()
        transpose(s, b)
        store_copy(s, b).start()
    for b in range(NBUF):
        s = (N_GROUPS - 1) * NBUF + b
        store_copy(s, b).wait()


@jax.jit
def _embed(indices_t, table_t):
    mesh = plsc.VectorSubcoreMesh(core_axis_name="c", subcore_axis_name="s")
    ka = pl.kernel(
        _transpose_body,
        out_type=jax.ShapeDtypeStruct((NPAIR, 128), jnp.float32),
        mesh=mesh,
        scratch_types=[
            pltpu.VMEM((TNBUF, D_MODEL, 128), jnp.float32),
            pltpu.VMEM((TNBUF, D_MODEL, 128), jnp.float32),
            pltpu.SemaphoreType.DMA((TNBUF,)),
            pltpu.SemaphoreType.DMA((TNBUF,)),
        ],
        compiler_params=pltpu.CompilerParams(needs_layout_passes=False),
    )
    tail = jnp.pad(table_t[:, NFULL * 128:].T, ((0, 0), (0, 64)))
    table2 = ka(table_t, tail)
    kb = pl.kernel(
        _gather_body,
        out_type=jax.ShapeDtypeStruct((SEQ, D_MODEL, BATCH), jnp.float32),
        mesh=mesh,
        scratch_types=[
            pltpu.VMEM((SEQ, BBLK), jnp.int32),
            pltpu.VMEM((NBUF, BBLK), jnp.int32),
            pltpu.VMEM((NBUF, BBLK), jnp.int32),
            pltpu.VMEM((NBUF, BBLK, 128), jnp.float32),
            pltpu.VMEM((NBUF, D_MODEL, BBLK), jnp.float32),
            pltpu.SemaphoreType.DMA((NBUF,)),
            pltpu.SemaphoreType.DMA((NBUF,)),
        ],
        compiler_params=pltpu.CompilerParams(needs_layout_passes=False),
    )
    return kb(indices_t, table2)


def kernel(indices, table):
    idx_t = indices.T.astype(jnp.int32)
    table_t = table.T
    out_t = _embed(idx_t, table_t)
    return jnp.transpose(out_t, (2, 0, 1))


# speedup vs baseline: 2.9364x; 1.4615x over previous
import jax
import jax.numpy as jnp
from jax import lax
from jax.experimental import pallas as pl
from jax.experimental.pallas import tpu as pltpu
from jax.experimental.pallas import tpu_sc as plsc

NC = 2
NS = 16
NW = NC * NS

BATCH = 4096
SEQ = 200
D_MODEL = 64
NPAIR = 500000
BBLK = BATCH // NW
NBUF = 4
N_GROUPS = SEQ // NBUF


NFULL = 1000000 // 128
NBLK_MAIN = NFULL // NW
NTAIL = NFULL - NBLK_MAIN * NW
TNBUF = 3


def _transpose_body(tt_hbm, tail_hbm, t2_hbm, slab_v, pair_v, gsems, ssems):
    t = lax.axis_index("s") * NC + lax.axis_index("c")
    iota = lax.iota(jnp.int32, 16)
    two_q = tuple(2 * (16 * qb + iota) for qb in range(4))
    qvs = tuple(16 * qb + iota for qb in range(4))

    def load_copy(ig, b):
        src = tt_hbm.at[:, pl.ds(ig * 128, 128)]
        return pltpu.make_async_copy(src, slab_v.at[b], gsems.at[b])

    def store_copy(ig, b):
        dst = t2_hbm.at[pl.ds(ig * 64, 64)]
        return pltpu.make_async_copy(pair_v.at[b], dst, ssems.at[b])

    def transpose(b, nqb):
        @plsc.parallel_loop(0, 128, 1, unroll=4)
        def fcol(c2):
            c2v = lax.bitwise_and(c2 + iota, 127)
            fv = lax.bitwise_and(c2v, 63)
            hv = lax.shift_right_logical(c2v, 6)
            for qb in range(nqb):
                vals = plsc.load_gather(slab_v.at[b], [fv, two_q[qb] + hv])
                plsc.store_scatter(pair_v.at[b], [qvs[qb], c2v], vals)

    def my_ig(n):
        return n * NW + t

    for b in range(TNBUF):
        load_copy(my_ig(b), b).start()

    for b in range(TNBUF):
        n = b
        load_copy(my_ig(n), b).wait()
        transpose(b, 4)
        store_copy(my_ig(n), b).start()
        load_copy(my_ig(n + TNBUF), b).start()

    def loop(n, _):
        for b in range(TNBUF):
            i = n * TNBUF + b
            load_copy(my_ig(i), b).wait()
            store_copy(my_ig(i - TNBUF), b).wait()
            transpose(b, 4)
            store_copy(my_ig(i), b).start()
            load_copy(my_ig(i + TNBUF), b).start()
        return ()

    lax.fori_loop(1, NBLK_MAIN // TNBUF - 1, loop, ())

    for b in range(TNBUF):
        i = (NBLK_MAIN // TNBUF - 1) * TNBUF + b
        load_copy(my_ig(i), b).wait()
        store_copy(my_ig(i - TNBUF), b).wait()
        transpose(b, 4)
        store_copy(my_ig(i), b).start()

    rem = NBLK_MAIN - (NBLK_MAIN // TNBUF) * TNBUF
    for b in range(rem):
        i = (NBLK_MAIN // TNBUF) * TNBUF + b
        load_copy(my_ig(i), b).start()
        load_copy(my_ig(i), b).wait()
        store_copy(my_ig(i - TNBUF), b).wait()
        transpose(b, 4)
        store_copy(my_ig(i), b).start()
    for b in range(TNBUF):
        i = NBLK_MAIN - TNBUF + b
        store_copy(my_ig(i), b).wait()

    @pl.when(t < NTAIL)
    def _():
        ig = NBLK_MAIN * NW + t
        load_copy(ig, 0).start()
        load_copy(ig, 0).wait()
        transpose(0, 4)
        store_copy(ig, 0).start()
        store_copy(ig, 0).wait()

    @pl.when(t == NTAIL)
    def _():
        pltpu.sync_copy(tail_hbm, slab_v.at[0])

        @plsc.parallel_loop(0, 128, 1, unroll=4)
        def fcol(c2):
            c2v = lax.bitwise_and(c2 + iota, 127)
            fv = lax.bitwise_and(c2v, 63)
            hv = lax.shift_right_logical(c2v, 6)
            for qb in range(2):
                vals = plsc.load_gather(
                    slab_v.at[0], [two_q[qb] + hv, fv])
                plsc.store_scatter(pair_v.at[0], [qvs[qb], c2v], vals)

        pltpu.sync_copy(pair_v.at[0].at[pl.ds(0, 32)],
                        t2_hbm.at[pl.ds(NFULL * 64, 32)])


def _gather_body(idxT_hbm, table2_hbm, outT_hbm, idx_v, pidx_v, off_v,
                 rows_v, trans_v, gsems, ssems):
    t = lax.axis_index("s") * NC + lax.axis_index("c")
    b0 = t * BBLK

    pltpu.sync_copy(idxT_hbm.at[:, pl.ds(b0, BBLK)], idx_v)

    iota = lax.iota(jnp.int32, 16)

    def compute_pidx(s, b):
        for k in range(BBLK // 16):
            v = idx_v[s, pl.ds(16 * k, 16)]
            pidx_v[b, pl.ds(16 * k, 16)] = lax.shift_right_logical(v, 1)

    def gather_copy(b):
        return pltpu.make_async_copy(
            table2_hbm.at[pidx_v.at[b]], rows_v.at[b], gsems.at[b])

    def store_copy(s, b):
        dst = outT_hbm.at[s].at[:, pl.ds(b0, BBLK)]
        return pltpu.make_async_copy(trans_v.at[b], dst, ssems.at[b])

    def transpose(s, b):
        for rb in range(BBLK // 16):
            off_v[b, pl.ds(16 * rb, 16)] = lax.shift_left(
                lax.bitwise_and(idx_v[s, pl.ds(16 * rb, 16)], 1), 6)

        @plsc.parallel_loop(0, D_MODEL, 1, unroll=8)
        def frow(f):
            feat = lax.bitwise_and(f + iota, D_MODEL - 1)
            for rb in range(BBLK // 16):
                rows = iota + 16 * rb
                offs = off_v[b, pl.ds(16 * rb, 16)]
                vals = plsc.load_gather(rows_v.at[b], [rows, offs + feat])
                plsc.store_scatter(trans_v.at[b], [feat, rows], vals)

    for b in range(NBUF):
        compute_pidx(b, b)
        gather_copy(b).start()

    for b in range(NBUF):
        s = b
        gather_copy(b).wait()
        transpose(s, b)
        store_copy(s, b).start()
        compute_pidx(s + NBUF, b)
        gather_copy(b).start()

    def group(g, _):
        for b in range(NBUF):
            s = g * NBUF + b
            gather_copy(b).wait()
            store_copy(s - NBUF, b).wait()
            transpose(s, b)
            store_copy(s, b).start()
            compute_pidx(s + NBUF, b)
            gather_copy(b).start()
        return ()

    lax.fori_loop(1, N_GROUPS - 1, group, ())

    for b in range(NBUF):
        s = (N_GROUPS - 1) * NBUF + b
        gather_copy(b).wait()
        store_copy(s - NBUF, b).wait()
        transpose(s, b)
        store_copy(s, b).start()
    for b in range(NBUF):
        s = (N_GROUPS - 1) * NBUF + b
        store_copy(s, b).wait()


@jax.jit
def _embed(indices_t, table_t, tail):
    mesh = plsc.VectorSubcoreMesh(core_axis_name="c", subcore_axis_name="s")
    ka = pl.kernel(
        _transpose_body,
        out_type=jax.ShapeDtypeStruct((NPAIR, 128), jnp.float32),
        mesh=mesh,
        scratch_types=[
            pltpu.VMEM((TNBUF, D_MODEL, 128), jnp.float32),
            pltpu.VMEM((TNBUF, D_MODEL, 128), jnp.float32),
            pltpu.SemaphoreType.DMA((TNBUF,)),
            pltpu.SemaphoreType.DMA((TNBUF,)),
        ],
        compiler_params=pltpu.CompilerParams(needs_layout_passes=False),
    )
    table2 = ka(table_t, tail)
    kb = pl.kernel(
        _gather_body,
        out_type=jax.ShapeDtypeStruct((SEQ, D_MODEL, BATCH), jnp.float32),
        mesh=mesh,
        scratch_types=[
            pltpu.VMEM((SEQ, BBLK), jnp.int32),
            pltpu.VMEM((NBUF, BBLK), jnp.int32),
            pltpu.VMEM((NBUF, BBLK), jnp.int32),
            pltpu.VMEM((NBUF, BBLK, 128), jnp.float32),
            pltpu.VMEM((NBUF, D_MODEL, BBLK), jnp.float32),
            pltpu.SemaphoreType.DMA((NBUF,)),
            pltpu.SemaphoreType.DMA((NBUF,)),
        ],
        compiler_params=pltpu.CompilerParams(needs_layout_passes=False),
    )
    return kb(indices_t, table2)


def kernel(indices, table):
    idx_t = indices.T.astype(jnp.int32)
    table_t = table.T
    tail = jnp.pad(table[NFULL * 128:, :], ((0, 0), (0, 64)))
    out_t = _embed(idx_t, table_t, tail)
    return jnp.transpose(out_t, (2, 0, 1))
